# baseline (speedup 1.0000x reference)
import jax
import jax.numpy as jnp
from jax import lax
from jax.experimental import pallas as pl
from jax.experimental.pallas import tpu as pltpu
from jax.experimental.pallas import tpu_sc as plsc

NC, NS, L = 2, 16, 16
NW = NC * NS
HID = 128
ROWS = 65
NF = 4
B, S = 4096, 200
BG = B // 128
SLAB = NF * 128
WS = 8
NWIN = S // WS
NGRP = 128 // L
TAB_WORDS = NF * ROWS * HID


def _body(ids_hbm, tab_hbm, out_hbm, tab_v, ids_v, blk_v, sem_tab, sem_ids,
          sem_out):
    wid = lax.axis_index("s") * NC + lax.axis_index("c")
    bg = wid
    b0 = bg * 128

    pltpu.async_copy(tab_hbm, tab_v, sem_tab).wait()

    def load_ids(w, slot):
        return pltpu.async_copy(
            ids_hbm.at[pl.ds(w * WS, WS), bg], ids_v.at[slot], sem_ids)

    load_ids(0, 0).wait()

    def win_body(w, _):
        slot = lax.rem(w, 2)
        s0 = w * WS

        @pl.when(w + 1 < NWIN)
        def _():
            load_ids(w + 1, 1 - slot)

        def grp_body(grp, _):
            gslot = lax.rem(grp, 2)

            @pl.when(grp >= 2)
            def _():
                pltpu.make_async_copy(
                    blk_v.at[0], out_hbm.at[pl.ds(0, L), pl.ds(0, WS)],
                    sem_out).wait()

            @plsc.parallel_loop(0, WS, unroll=2)
            def s_body(sl):
                vecs = [
                    ids_v[slot, sl, pl.ds(f * 128 + grp * L, L)]
                    for f in range(NF)
                ]
                for j in range(L):
                    o0 = vecs[0][j]
                    o1 = vecs[1][j]
                    o2 = vecs[2][j]
                    o3 = vecs[3][j]
                    for c in range(HID // L):
                        t0 = tab_v[pl.ds(o0 + c * L, L)]
                        t1 = tab_v[pl.ds(o1 + c * L, L)]
                        t2 = tab_v[pl.ds(o2 + c * L, L)]
                        t3 = tab_v[pl.ds(o3 + c * L, L)]
                        blk_v[gslot, j, sl, pl.ds(c * L, L)] = \
                            (t0 + t1) + (t2 + t3)

            for j in range(L):
                pltpu.async_copy(
                    blk_v.at[gslot, j],
                    out_hbm.at[b0 + grp * L + j, pl.ds(s0, WS)], sem_out)
            return 0

        lax.fori_loop(0, NGRP, grp_body, 0)

        for _ in range(2):
            pltpu.make_async_copy(
                blk_v.at[0], out_hbm.at[pl.ds(0, L), pl.ds(0, WS)],
                sem_out).wait()

        @pl.when(w + 1 < NWIN)
        def _():
            pltpu.make_async_copy(
                ids_v.at[0], ids_hbm.at[pl.ds(0, WS), 0], sem_ids).wait()
        return 0

    lax.fori_loop(0, NWIN, win_body, 0)


@jax.jit
def _run(offs, tab_flat):
    mesh = plsc.VectorSubcoreMesh(core_axis_name="c", subcore_axis_name="s",
                                  num_cores=NC, num_subcores=NS)
    return pl.kernel(
        _body,
        out_type=jax.ShapeDtypeStruct((B, S, HID), jnp.float32),
        mesh=mesh,
        scratch_types=[
            pltpu.VMEM((TAB_WORDS,), jnp.float32),
            pltpu.VMEM((2, WS, SLAB), jnp.int32),
            pltpu.VMEM((2, L, WS, HID), jnp.float32),
            pltpu.SemaphoreType.DMA,
            pltpu.SemaphoreType.DMA,
            pltpu.SemaphoreType.DMA,
        ],
        compiler_params=pltpu.CompilerParams(needs_layout_passes=False),
    )(offs, tab_flat)


def kernel(noise_ids, W0, W1, W2, W3):
    featbase = jnp.array([i * ROWS * HID for i in range(NF)], jnp.int32)
    offs = noise_ids * HID + featbase
    offs_sb = (offs.reshape(BG, 128, S, NF)
               .transpose(2, 0, 3, 1)
               .reshape(S, BG, SLAB))
    tab_flat = jnp.concatenate([W0, W1, W2, W3], axis=0).reshape(-1)
    return _run(offs_sb, tab_flat)
